# baseline (speedup 1.0000x reference)
import functools

import numpy as np
import jax
import jax.numpy as jnp
from jax import lax
from jax.experimental import pallas as pl
from jax.experimental.pallas import tpu as pltpu
from jax.experimental.pallas import tpu_sc as plsc

N = 65536
D = 64
_A = np.float32(0.98)
_OMA = np.float32(1.0 - 0.98)
_H = np.sqrt(np.float32(1.0 - 0.98 ** 2))
_H2 = np.float32(_H * _H)
_LO = np.nextafter(np.float32(-1.0), np.float32(0.0), dtype=np.float32)
_DELTA = np.float32(np.float32(1.0) - _LO)
_SQRT2 = np.float32(np.sqrt(2.0))


_R0 = (13, 15, 26, 6)
_R1 = (17, 29, 16, 24)


def _rotl(x, r):
  return (x << jnp.uint32(r)) | (x >> jnp.uint32(32 - r))


def _tf_group(x0, x1, rots):
  for r in rots:
    x0 = x0 + x1
    x1 = _rotl(x1, r)
    x1 = x0 ^ x1
  return x0, x1


def _threefry2x32(ks0, ks1, c0, c1):
  ks2 = ks0 ^ ks1 ^ jnp.uint32(0x1BD11BDA)
  x0 = c0 + ks0
  x1 = c1 + ks1
  x0, x1 = _tf_group(x0, x1, _R0)
  x0 = x0 + ks1
  x1 = x1 + ks2 + jnp.uint32(1)
  x0, x1 = _tf_group(x0, x1, _R1)
  x0 = x0 + ks2
  x1 = x1 + ks0 + jnp.uint32(2)
  x0, x1 = _tf_group(x0, x1, _R0)
  x0 = x0 + ks0
  x1 = x1 + ks1 + jnp.uint32(3)
  x0, x1 = _tf_group(x0, x1, _R1)
  x0 = x0 + ks1
  x1 = x1 + ks2 + jnp.uint32(4)
  x0, x1 = _tf_group(x0, x1, _R0)
  x0 = x0 + ks2
  x1 = x1 + ks0 + jnp.uint32(5)
  return x0, x1


def _random_bits_32(ks0, ks1, linear_index_u32):
  o0, o1 = _threefry2x32(ks0, ks1, jnp.uint32(0), linear_index_u32)
  return o0 ^ o1


def _bits_to_unit_float(bits):
  fb = (bits >> jnp.uint32(9)) | jnp.uint32(0x3F800000)
  return lax.bitcast_convert_type(fb, jnp.float32) - jnp.float32(1.0)


_ERFINV_P = (3.20952757e-04, -5.89907452e-03, 4.35519785e-02, -1.61653791e-01,
             3.06063571e-01, -2.90852599e-01, 3.82224641e-01, -3.53325403e-02,
             8.88815239e-01)


def _erfinv(x):
  w = -jnp.log((jnp.float32(1.0) - x) * (jnp.float32(1.0) + x))
  s = jnp.sqrt(w)
  p = jnp.float32(_ERFINV_P[0])
  for c in _ERFINV_P[1:]:
    p = p * s + jnp.float32(c)
  return p * x


_BCS = 16384


def _stats_body(xt_ref, w_ref, mu_ref, l_ref, m_sc, mu_sc, s_sc):
  i = pl.program_id(0)

  @pl.when(i == 0)
  def _():
    m_sc[...] = jnp.zeros_like(m_sc)
    mu_sc[...] = jnp.zeros_like(mu_sc)
    s_sc[0, 0] = jnp.float32(0.0)

  xt = xt_ref[...]
  w = w_ref[...]
  y = xt * w
  m_sc[...] += lax.dot_general(y, xt, (((1,), (1,)), ((), ())),
                               preferred_element_type=jnp.float32)
  mu_sc[...] += jnp.sum(y, axis=1, keepdims=True)
  s_sc[0, 0] += jnp.sum(w)

  @pl.when(i == pl.num_programs(0) - 1)
  def _():
    mu = mu_sc[...]
    s = s_sc[0, 0]
    outer = lax.dot_general(mu, mu, (((1,), (1,)), ((), ())),
                            preferred_element_type=jnp.float32)
    cov = (m_sc[...] - (jnp.float32(2.0) - s) * outer) * _H2
    rows = lax.broadcasted_iota(jnp.int32, (D, D), 0)
    cols = lax.broadcasted_iota(jnp.int32, (D, D), 1)
    rcol = lax.broadcasted_iota(jnp.int32, (D, 1), 0)

    def chol_step(j, lmat):
      lrow = jnp.sum(jnp.where(rows == j, lmat, 0.0), axis=0,
                     keepdims=True)
      v = jnp.sum(lmat * lrow, axis=1, keepdims=True)
      acol = jnp.sum(jnp.where(cols == j, cov, 0.0), axis=1,
                     keepdims=True)
      c = acol - v
      diag = jnp.sum(jnp.where(rcol == j, c, 0.0))
      newcol = c / jnp.sqrt(diag)
      return jnp.where((cols == j) & (rows >= j), newcol, lmat)

    l_ref[...] = lax.fori_loop(0, D, chol_step,
                               jnp.zeros((D, D), jnp.float32))
    mu_ref[...] = mu


def _stats_chol(particles_t, w_row):
  return pl.pallas_call(
      _stats_body,
      grid=(N // _BCS,),
      in_specs=[
          pl.BlockSpec((D, _BCS), lambda i: (0, i)),
          pl.BlockSpec((1, _BCS), lambda i: (0, i)),
      ],
      out_specs=[
          pl.BlockSpec((D, 1), lambda i: (0, 0)),
          pl.BlockSpec((D, D), lambda i: (0, 0)),
      ],
      out_shape=[
          jax.ShapeDtypeStruct((D, 1), jnp.float32),
          jax.ShapeDtypeStruct((D, D), jnp.float32),
      ],
      scratch_shapes=[
          pltpu.VMEM((D, D), jnp.float32),
          pltpu.VMEM((D, 1), jnp.float32),
          pltpu.SMEM((1, 1), jnp.float32),
      ],
  )(particles_t, w_row)


_NW = 32
_SPW = N // _NW
_CH = 128
_NCH = _SPW // _CH


def _sc_body(pc_hbm, x_hbm, k_hbm, plast_hbm, out_hbm,
             pc_v, idx_v, k_v, plast_v, buf0, buf1, gsem, wsem):
  wid = lax.axis_index("s") * 2 + lax.axis_index("c")
  base = wid * _SPW
  pltpu.sync_copy(pc_hbm, pc_v)
  pltpu.sync_copy(k_hbm, k_v)
  pltpu.sync_copy(plast_hbm, plast_v)
  k0 = k_v[pl.ds(0, 16)]
  k1 = k_v[pl.ds(16, 16)]
  plast = plast_v[...]

  def qbody(c, carry):
    e = jnp.full((16,), base, jnp.int32) + c * 16 + lax.iota(jnp.int32, 16)
    bits = _random_bits_32(k0, k1, e.astype(jnp.uint32))
    u = _bits_to_unit_float(bits)
    r = plast * (jnp.float32(1.0) - u)

    def sstep(t, lh):
      lo, hi = lh
      mid = (lo + hi) >> 1
      cm = plsc.load_gather(pc_v, [mid])
      pred = cm < r
      return (jnp.where(pred, mid + 1, lo), jnp.where(pred, hi, mid))

    lo, _ = lax.fori_loop(0, 17, sstep,
                          (jnp.zeros((16,), jnp.int32),
                           jnp.full((16,), N, jnp.int32)))
    idx_v[pl.ds(c * 16, 16)] = lo
    return carry

  lax.fori_loop(0, _SPW // 16, qbody, 0)

  bufs = (buf0, buf1)
  gathers = [None] * _NCH
  writes = [None] * _NCH
  gathers[0] = pltpu.async_copy(x_hbm.at[idx_v.at[pl.ds(0, _CH)]], buf0, gsem)
  for g in range(_NCH):
    gathers[g].wait()
    if g + 1 < _NCH:
      if g >= 1:
        writes[g - 1].wait()
      gathers[g + 1] = pltpu.async_copy(
          x_hbm.at[idx_v.at[pl.ds((g + 1) * _CH, _CH)]],
          bufs[(g + 1) % 2], gsem)
    half = base // (N // 2)
    r0 = base - half * (N // 2) + g * _CH
    writes[g] = pltpu.async_copy(
        bufs[g % 2], out_hbm.at[pl.ds(r0, _CH), pl.ds(half * D, D)], wsem)
  writes[_NCH - 2].wait()
  writes[_NCH - 1].wait()


def _sc_sample(p_cuml, particles, kvec, plast_vec):
  mesh = plsc.VectorSubcoreMesh(core_axis_name="c", subcore_axis_name="s")
  f = pl.kernel(
      _sc_body,
      out_type=jax.ShapeDtypeStruct((N // 2, 2 * D), jnp.float32),
      mesh=mesh,
      scratch_types=[
          pltpu.VMEM((N,), jnp.float32),
          pltpu.VMEM((_SPW,), jnp.int32),
          pltpu.VMEM((32,), jnp.uint32),
          pltpu.VMEM((16,), jnp.float32),
          pltpu.VMEM((_CH, D), jnp.float32),
          pltpu.VMEM((_CH, D), jnp.float32),
          pltpu.SemaphoreType.DMA,
          pltpu.SemaphoreType.DMA,
      ],
      compiler_params=pltpu.CompilerParams(needs_layout_passes=False,
                                           use_tc_tiling_on_sc=False),
  )
  return f(p_cuml, particles, kvec, plast_vec)


_BC2 = 256


def _noise_final_body(k_ref, mu2_ref, lbig_ref, chosen_ref, out_ref, s_sc):
  j = pl.program_id(0)
  h = pl.program_id(1)

  @pl.when(h == 0)
  def _():
    ks0 = k_ref[0]
    ks1 = k_ref[1]
    rows = lax.broadcasted_iota(jnp.int32, (_BC2, 2 * D), 0)
    cols = lax.broadcasted_iota(jnp.int32, (_BC2, 2 * D), 1)
    e = ((j * _BC2 + rows) * D
         + ((cols & (D - 1)) + (cols >> 6) * jnp.int32(N * D // 2)))
    bits = _random_bits_32(ks0, ks1, e.astype(jnp.uint32))
    f = _bits_to_unit_float(bits)
    u = jnp.maximum(jnp.float32(_LO),
                    f * jnp.float32(_DELTA) + jnp.float32(_LO))
    z2 = jnp.float32(_SQRT2) * _erfinv(u)
    noise2 = lax.dot_general(z2, lbig_ref[...], (((1,), (0,)), ((), ())),
                             preferred_element_type=jnp.float32)
    s_sc[...] = (jnp.float32(_A) * chosen_ref[...]
                 + (jnp.float32(_OMA) * mu2_ref[...] + noise2))

  s2 = s_sc[...]
  half = jnp.where(h == 0, s2[:, :D], s2[:, D:])
  out_ref[...] = half.T


def _noise_final(kvec2, mu2, lbig, chosen2):
  nb = N // 2 // _BC2
  return pl.pallas_call(
      _noise_final_body,
      grid=(nb, 2),
      in_specs=[
          pl.BlockSpec(memory_space=pltpu.SMEM),
          pl.BlockSpec((1, 2 * D), lambda j, h: (0, 0)),
          pl.BlockSpec((2 * D, 2 * D), lambda j, h: (0, 0)),
          pl.BlockSpec((_BC2, 2 * D), lambda j, h: (j, 0)),
      ],
      out_specs=pl.BlockSpec((D, _BC2), lambda j, h: (0, h * nb + j)),
      out_shape=jax.ShapeDtypeStruct((D, N), jnp.float32),
      scratch_shapes=[pltpu.VMEM((_BC2, 2 * D), jnp.float32)],
      compiler_params=pltpu.CompilerParams(
          dimension_semantics=("arbitrary", "arbitrary")),
  )(kvec2, mu2, lbig, chosen2)


def kernel(key, particles_locations, weights):
  key1, sub1 = jax.random.split(key)
  _, sub2 = jax.random.split(key1)
  kd1 = jax.random.key_data(sub1).astype(jnp.uint32)
  kd2 = jax.random.key_data(sub2).astype(jnp.uint32)

  p_cuml = jnp.cumsum(weights)
  plast_vec = jnp.full((16,), p_cuml[-1], jnp.float32)
  kvec = jnp.concatenate([jnp.full((16,), kd1[0], jnp.uint32),
                          jnp.full((16,), kd1[1], jnp.uint32)])

  chosen = _sc_sample(p_cuml, particles_locations, kvec, plast_vec)
  mu_col, lmat = _stats_chol(particles_locations.T, weights.reshape(1, N))
  mu = mu_col.T
  mu2 = jnp.concatenate([mu, mu], axis=1)
  lt = lmat.T
  zpad = jnp.zeros((D, D), jnp.float32)
  lbig = jnp.block([[lt, zpad], [zpad, lt]])
  out_t = _noise_final(kd2, mu2, lbig, chosen)
  out = out_t.T
  new_weights = jnp.full((N,), np.float32(1.0 / N), jnp.float32)
  return out, new_weights


# speedup vs baseline: 1.4679x; 1.4679x over previous
import functools

import numpy as np
import jax
import jax.numpy as jnp
from jax import lax
from jax.experimental import pallas as pl
from jax.experimental.pallas import tpu as pltpu
from jax.experimental.pallas import tpu_sc as plsc

N = 65536
D = 64
_A = np.float32(0.98)
_OMA = np.float32(1.0 - 0.98)
_H = np.sqrt(np.float32(1.0 - 0.98 ** 2))
_H2 = np.float32(_H * _H)
_LO = np.nextafter(np.float32(-1.0), np.float32(0.0), dtype=np.float32)
_DELTA = np.float32(np.float32(1.0) - _LO)
_SQRT2 = np.float32(np.sqrt(2.0))


_R0 = (13, 15, 26, 6)
_R1 = (17, 29, 16, 24)


def _rotl(x, r):
  return (x << jnp.uint32(r)) | (x >> jnp.uint32(32 - r))


def _tf_group(x0, x1, rots):
  for r in rots:
    x0 = x0 + x1
    x1 = _rotl(x1, r)
    x1 = x0 ^ x1
  return x0, x1


def _threefry2x32(ks0, ks1, c0, c1):
  ks2 = ks0 ^ ks1 ^ jnp.uint32(0x1BD11BDA)
  x0 = c0 + ks0
  x1 = c1 + ks1
  x0, x1 = _tf_group(x0, x1, _R0)
  x0 = x0 + ks1
  x1 = x1 + ks2 + jnp.uint32(1)
  x0, x1 = _tf_group(x0, x1, _R1)
  x0 = x0 + ks2
  x1 = x1 + ks0 + jnp.uint32(2)
  x0, x1 = _tf_group(x0, x1, _R0)
  x0 = x0 + ks0
  x1 = x1 + ks1 + jnp.uint32(3)
  x0, x1 = _tf_group(x0, x1, _R1)
  x0 = x0 + ks1
  x1 = x1 + ks2 + jnp.uint32(4)
  x0, x1 = _tf_group(x0, x1, _R0)
  x0 = x0 + ks2
  x1 = x1 + ks0 + jnp.uint32(5)
  return x0, x1


def _random_bits_32(ks0, ks1, linear_index_u32):
  o0, o1 = _threefry2x32(ks0, ks1, jnp.uint32(0), linear_index_u32)
  return o0 ^ o1


def _bits_to_unit_float(bits):
  fb = (bits >> jnp.uint32(9)) | jnp.uint32(0x3F800000)
  return lax.bitcast_convert_type(fb, jnp.float32) - jnp.float32(1.0)


_ERFINV_P = (3.20952757e-04, -5.89907452e-03, 4.35519785e-02, -1.61653791e-01,
             3.06063571e-01, -2.90852599e-01, 3.82224641e-01, -3.53325403e-02,
             8.88815239e-01)


def _erfinv(x):
  w = -jnp.log((jnp.float32(1.0) - x) * (jnp.float32(1.0) + x))
  s = jnp.sqrt(w)
  p = jnp.float32(_ERFINV_P[0])
  for c in _ERFINV_P[1:]:
    p = p * s + jnp.float32(c)
  return p * x


_BCS = 16384


def _stats_body(xt_ref, w_ref, mu_ref, l_ref, m_sc, mu_sc, s_sc):
  i = pl.program_id(0)

  @pl.when(i == 0)
  def _():
    m_sc[...] = jnp.zeros_like(m_sc)
    mu_sc[...] = jnp.zeros_like(mu_sc)
    s_sc[0, 0] = jnp.float32(0.0)

  xt = xt_ref[...]
  w = w_ref[...]
  y = xt * w
  m_sc[...] += lax.dot_general(y, xt, (((1,), (1,)), ((), ())),
                               preferred_element_type=jnp.float32)
  mu_sc[...] += jnp.sum(y, axis=1, keepdims=True)
  s_sc[0, 0] += jnp.sum(w)

  @pl.when(i == pl.num_programs(0) - 1)
  def _():
    mu = mu_sc[...]
    s = s_sc[0, 0]
    outer = lax.dot_general(mu, mu, (((1,), (1,)), ((), ())),
                            preferred_element_type=jnp.float32)
    cov = (m_sc[...] - (jnp.float32(2.0) - s) * outer) * _H2
    rows = lax.broadcasted_iota(jnp.int32, (D, D), 0)
    cols = lax.broadcasted_iota(jnp.int32, (D, D), 1)
    rcol = lax.broadcasted_iota(jnp.int32, (D, 1), 0)

    def chol_step(j, lmat):
      lrow = jnp.sum(jnp.where(rows == j, lmat, 0.0), axis=0,
                     keepdims=True)
      v = jnp.sum(lmat * lrow, axis=1, keepdims=True)
      acol = jnp.sum(jnp.where(cols == j, cov, 0.0), axis=1,
                     keepdims=True)
      c = acol - v
      diag = jnp.sum(jnp.where(rcol == j, c, 0.0))
      newcol = c / jnp.sqrt(diag)
      return jnp.where((cols == j) & (rows >= j), newcol, lmat)

    l_ref[...] = lax.fori_loop(0, D, chol_step,
                               jnp.zeros((D, D), jnp.float32))
    mu_ref[...] = mu


def _stats_chol(particles_t, w_row):
  return pl.pallas_call(
      _stats_body,
      grid=(N // _BCS,),
      in_specs=[
          pl.BlockSpec((D, _BCS), lambda i: (0, i)),
          pl.BlockSpec((1, _BCS), lambda i: (0, i)),
      ],
      out_specs=[
          pl.BlockSpec((D, 1), lambda i: (0, 0)),
          pl.BlockSpec((D, D), lambda i: (0, 0)),
      ],
      out_shape=[
          jax.ShapeDtypeStruct((D, 1), jnp.float32),
          jax.ShapeDtypeStruct((D, D), jnp.float32),
      ],
      scratch_shapes=[
          pltpu.VMEM((D, D), jnp.float32),
          pltpu.VMEM((D, 1), jnp.float32),
          pltpu.SMEM((1, 1), jnp.float32),
      ],
  )(particles_t, w_row)


_NW = 32
_SPW = N // _NW
_CH = 128
_NCH = _SPW // _CH


def _sc_body(pc_hbm, x_hbm, k_hbm, plast_hbm, out_hbm,
             pc_v, idx_v, k_v, plast_v, buf0, buf1, gsem, wsem):
  wid = lax.axis_index("s") * 2 + lax.axis_index("c")
  base = wid * _SPW
  pltpu.sync_copy(pc_hbm, pc_v)
  pltpu.sync_copy(k_hbm, k_v)
  pltpu.sync_copy(plast_hbm, plast_v)
  k0 = k_v[pl.ds(0, 16)]
  k1 = k_v[pl.ds(16, 16)]
  plast = plast_v[...]

  def qbody(c, carry):
    e = jnp.full((16,), base, jnp.int32) + c * 16 + lax.iota(jnp.int32, 16)
    bits = _random_bits_32(k0, k1, e.astype(jnp.uint32))
    u = _bits_to_unit_float(bits)
    r = plast * (jnp.float32(1.0) - u)

    def sstep(t, lh):
      lo, hi = lh
      mid = (lo + hi) >> 1
      cm = plsc.load_gather(pc_v, [mid])
      pred = cm < r
      return (jnp.where(pred, mid + 1, lo), jnp.where(pred, hi, mid))

    lo, _ = lax.fori_loop(0, 17, sstep,
                          (jnp.zeros((16,), jnp.int32),
                           jnp.full((16,), N, jnp.int32)))
    idx_v[pl.ds(c * 16, 16)] = lo
    return carry

  lax.fori_loop(0, _SPW // 16, qbody, 0)

  bufs = (buf0, buf1)
  gathers = [None] * _NCH
  writes = [None] * _NCH
  gathers[0] = pltpu.async_copy(x_hbm.at[idx_v.at[pl.ds(0, _CH)]], buf0, gsem)
  for g in range(_NCH):
    gathers[g].wait()
    if g + 1 < _NCH:
      if g >= 1:
        writes[g - 1].wait()
      gathers[g + 1] = pltpu.async_copy(
          x_hbm.at[idx_v.at[pl.ds((g + 1) * _CH, _CH)]],
          bufs[(g + 1) % 2], gsem)
    half = base // (N // 2)
    r0 = base - half * (N // 2) + g * _CH
    writes[g] = pltpu.async_copy(
        bufs[g % 2], out_hbm.at[pl.ds(r0, _CH), pl.ds(half * D, D)], wsem)
  writes[_NCH - 2].wait()
  writes[_NCH - 1].wait()


def _sc_sample(p_cuml, particles, kvec, plast_vec):
  mesh = plsc.VectorSubcoreMesh(core_axis_name="c", subcore_axis_name="s")
  f = pl.kernel(
      _sc_body,
      out_type=jax.ShapeDtypeStruct((N // 2, 2 * D), jnp.float32),
      mesh=mesh,
      scratch_types=[
          pltpu.VMEM((N,), jnp.float32),
          pltpu.VMEM((_SPW,), jnp.int32),
          pltpu.VMEM((32,), jnp.uint32),
          pltpu.VMEM((16,), jnp.float32),
          pltpu.VMEM((_CH, D), jnp.float32),
          pltpu.VMEM((_CH, D), jnp.float32),
          pltpu.SemaphoreType.DMA,
          pltpu.SemaphoreType.DMA,
      ],
      compiler_params=pltpu.CompilerParams(needs_layout_passes=False,
                                           use_tc_tiling_on_sc=False),
  )
  return f(p_cuml, particles, kvec, plast_vec)


_BC2 = 256


def _noise_body(k_ref, mu2_ref, lbig_ref, out_ref):
  i = pl.program_id(0)
  ks0 = k_ref[0]
  ks1 = k_ref[1]
  rows = lax.broadcasted_iota(jnp.int32, (_BC2, 2 * D), 0)
  cols = lax.broadcasted_iota(jnp.int32, (_BC2, 2 * D), 1)
  e = ((i * _BC2 + rows) * D
       + ((cols & (D - 1)) + (cols >> 6) * jnp.int32(N * D // 2)))
  bits = _random_bits_32(ks0, ks1, e.astype(jnp.uint32))
  f = _bits_to_unit_float(bits)
  u = jnp.maximum(jnp.float32(_LO), f * jnp.float32(_DELTA) + jnp.float32(_LO))
  z2 = jnp.float32(_SQRT2) * _erfinv(u)
  noise2 = lax.dot_general(z2, lbig_ref[...], (((1,), (0,)), ((), ())),
                           preferred_element_type=jnp.float32)
  out_ref[...] = jnp.float32(_OMA) * mu2_ref[...] + noise2


def _noise(kvec2, mu2, lbig):
  return pl.pallas_call(
      _noise_body,
      grid=(N // (2 * _BC2),),
      in_specs=[
          pl.BlockSpec(memory_space=pltpu.SMEM),
          pl.BlockSpec((1, 2 * D), lambda i: (0, 0)),
          pl.BlockSpec((2 * D, 2 * D), lambda i: (0, 0)),
      ],
      out_specs=pl.BlockSpec((_BC2, 2 * D), lambda i: (i, 0)),
      out_shape=jax.ShapeDtypeStruct((N // 2, 2 * D), jnp.float32),
      compiler_params=pltpu.CompilerParams(
          dimension_semantics=("arbitrary",)),
  )(kvec2, mu2, lbig)


_BF = 2048


def _final_body(chosen_ref, noise_ref, out_ref):
  h = pl.program_id(1)
  s = jnp.float32(_A) * chosen_ref[...] + noise_ref[...]
  half = jnp.where(h == 0, s[:, :D], s[:, D:])
  out_ref[...] = half.T


def _final(chosen2, noise2):
  nb = N // 2 // _BF
  return pl.pallas_call(
      _final_body,
      grid=(nb, 2),
      in_specs=[
          pl.BlockSpec((_BF, 2 * D), lambda j, h: (j, 0)),
          pl.BlockSpec((_BF, 2 * D), lambda j, h: (j, 0)),
      ],
      out_specs=pl.BlockSpec((D, _BF), lambda j, h: (0, h * nb + j)),
      out_shape=jax.ShapeDtypeStruct((D, N), jnp.float32),
      compiler_params=pltpu.CompilerParams(
          dimension_semantics=("arbitrary", "arbitrary")),
  )(chosen2, noise2)


def kernel(key, particles_locations, weights):
  key1, sub1 = jax.random.split(key)
  _, sub2 = jax.random.split(key1)
  kd1 = jax.random.key_data(sub1).astype(jnp.uint32)
  kd2 = jax.random.key_data(sub2).astype(jnp.uint32)

  p_cuml = jnp.cumsum(weights)
  plast_vec = jnp.full((16,), p_cuml[-1], jnp.float32)
  kvec = jnp.concatenate([jnp.full((16,), kd1[0], jnp.uint32),
                          jnp.full((16,), kd1[1], jnp.uint32)])

  chosen = _sc_sample(p_cuml, particles_locations, kvec, plast_vec)
  mu_col, lmat = _stats_chol(particles_locations.T, weights.reshape(1, N))
  mu = mu_col.T
  mu2 = jnp.concatenate([mu, mu], axis=1)
  lt = lmat.T
  zpad = jnp.zeros((D, D), jnp.float32)
  lbig = jnp.block([[lt, zpad], [zpad, lt]])
  noise2 = _noise(kd2, mu2, lbig)
  out_t = _final(chosen, noise2)
  out = out_t.T
  new_weights = jnp.full((N,), np.float32(1.0 / N), jnp.float32)
  return out, new_weights


# speedup vs baseline: 1.5222x; 1.0370x over previous
import functools

import numpy as np
import jax
import jax.numpy as jnp
from jax import lax
from jax.experimental import pallas as pl
from jax.experimental.pallas import tpu as pltpu
from jax.experimental.pallas import tpu_sc as plsc

N = 65536
D = 64
_A = np.float32(0.98)
_OMA = np.float32(1.0 - 0.98)
_H = np.sqrt(np.float32(1.0 - 0.98 ** 2))
_H2 = np.float32(_H * _H)
_LO = np.nextafter(np.float32(-1.0), np.float32(0.0), dtype=np.float32)
_DELTA = np.float32(np.float32(1.0) - _LO)
_SQRT2 = np.float32(np.sqrt(2.0))


_R0 = (13, 15, 26, 6)
_R1 = (17, 29, 16, 24)


def _rotl(x, r):
  return (x << jnp.uint32(r)) | (x >> jnp.uint32(32 - r))


def _tf_group(x0, x1, rots):
  for r in rots:
    x0 = x0 + x1
    x1 = _rotl(x1, r)
    x1 = x0 ^ x1
  return x0, x1


def _threefry2x32(ks0, ks1, c0, c1):
  ks2 = ks0 ^ ks1 ^ jnp.uint32(0x1BD11BDA)
  x0 = c0 + ks0
  x1 = c1 + ks1
  x0, x1 = _tf_group(x0, x1, _R0)
  x0 = x0 + ks1
  x1 = x1 + ks2 + jnp.uint32(1)
  x0, x1 = _tf_group(x0, x1, _R1)
  x0 = x0 + ks2
  x1 = x1 + ks0 + jnp.uint32(2)
  x0, x1 = _tf_group(x0, x1, _R0)
  x0 = x0 + ks0
  x1 = x1 + ks1 + jnp.uint32(3)
  x0, x1 = _tf_group(x0, x1, _R1)
  x0 = x0 + ks1
  x1 = x1 + ks2 + jnp.uint32(4)
  x0, x1 = _tf_group(x0, x1, _R0)
  x0 = x0 + ks2
  x1 = x1 + ks0 + jnp.uint32(5)
  return x0, x1


def _random_bits_32(ks0, ks1, linear_index_u32):
  o0, o1 = _threefry2x32(ks0, ks1, jnp.uint32(0), linear_index_u32)
  return o0 ^ o1


def _bits_to_unit_float(bits):
  fb = (bits >> jnp.uint32(9)) | jnp.uint32(0x3F800000)
  return lax.bitcast_convert_type(fb, jnp.float32) - jnp.float32(1.0)


_ERFINV_P = (3.20952757e-04, -5.89907452e-03, 4.35519785e-02, -1.61653791e-01,
             3.06063571e-01, -2.90852599e-01, 3.82224641e-01, -3.53325403e-02,
             8.88815239e-01)


def _erfinv(x):
  w = -jnp.log((jnp.float32(1.0) - x) * (jnp.float32(1.0) + x))
  s = jnp.sqrt(w)
  p = jnp.float32(_ERFINV_P[0])
  for c in _ERFINV_P[1:]:
    p = p * s + jnp.float32(c)
  return p * x


_BCS = 16384


def _stats_body(xt_ref, w_ref, mu_ref, l_ref, m_sc, mu_sc, s_sc):
  i = pl.program_id(0)

  @pl.when(i == 0)
  def _():
    m_sc[...] = jnp.zeros_like(m_sc)
    mu_sc[...] = jnp.zeros_like(mu_sc)
    s_sc[0, 0] = jnp.float32(0.0)

  xt = xt_ref[...]
  w = w_ref[...]
  y = xt * w
  m_sc[...] += lax.dot_general(y, xt, (((1,), (1,)), ((), ())),
                               preferred_element_type=jnp.float32)
  mu_sc[...] += jnp.sum(y, axis=1, keepdims=True)
  s_sc[0, 0] += jnp.sum(w)

  @pl.when(i == pl.num_programs(0) - 1)
  def _():
    mu = mu_sc[...]
    s = s_sc[0, 0]
    outer = lax.dot_general(mu, mu, (((1,), (1,)), ((), ())),
                            preferred_element_type=jnp.float32)
    cov = (m_sc[...] - (jnp.float32(2.0) - s) * outer) * _H2
    rows = lax.broadcasted_iota(jnp.int32, (D, D), 0)
    cols = lax.broadcasted_iota(jnp.int32, (D, D), 1)
    rcol = lax.broadcasted_iota(jnp.int32, (D, 1), 0)

    def chol_step(j, lmat):
      lrow = jnp.sum(jnp.where(rows == j, lmat, 0.0), axis=0,
                     keepdims=True)
      v = jnp.sum(lmat * lrow, axis=1, keepdims=True)
      acol = jnp.sum(jnp.where(cols == j, cov, 0.0), axis=1,
                     keepdims=True)
      c = acol - v
      diag = jnp.sum(jnp.where(rcol == j, c, 0.0))
      newcol = c / jnp.sqrt(diag)
      return jnp.where((cols == j) & (rows >= j), newcol, lmat)

    l_ref[...] = lax.fori_loop(0, D, chol_step,
                               jnp.zeros((D, D), jnp.float32))
    mu_ref[...] = mu


def _stats_chol(particles_t, w_row):
  return pl.pallas_call(
      _stats_body,
      grid=(N // _BCS,),
      in_specs=[
          pl.BlockSpec((D, _BCS), lambda i: (0, i)),
          pl.BlockSpec((1, _BCS), lambda i: (0, i)),
      ],
      out_specs=[
          pl.BlockSpec((D, 1), lambda i: (0, 0)),
          pl.BlockSpec((D, D), lambda i: (0, 0)),
      ],
      out_shape=[
          jax.ShapeDtypeStruct((D, 1), jnp.float32),
          jax.ShapeDtypeStruct((D, D), jnp.float32),
      ],
      scratch_shapes=[
          pltpu.VMEM((D, D), jnp.float32),
          pltpu.VMEM((D, 1), jnp.float32),
          pltpu.SMEM((1, 1), jnp.float32),
      ],
  )(particles_t, w_row)


_NW = 32
_SPW = N // _NW
_CH = 128
_NCH = _SPW // _CH


def _sc_body(pc_hbm, x_hbm, k_hbm, plast_hbm, out_hbm,
             pc_v, idx_v, k_v, plast_v, buf0, buf1, gsem, wsem):
  wid = lax.axis_index("s") * 2 + lax.axis_index("c")
  base = wid * _SPW
  pltpu.sync_copy(pc_hbm, pc_v)
  pltpu.sync_copy(k_hbm, k_v)
  pltpu.sync_copy(plast_hbm, plast_v)
  k0 = k_v[pl.ds(0, 16)]
  k1 = k_v[pl.ds(16, 16)]
  plast = plast_v[...]

  def qbody(c, carry):
    e = jnp.full((16,), base, jnp.int32) + c * 16 + lax.iota(jnp.int32, 16)
    bits = _random_bits_32(k0, k1, e.astype(jnp.uint32))
    u = _bits_to_unit_float(bits)
    r = plast * (jnp.float32(1.0) - u)

    def sstep(t, lh):
      lo, hi = lh
      mid = (lo + hi) >> 1
      cm = plsc.load_gather(pc_v, [mid])
      pred = cm < r
      return (jnp.where(pred, mid + 1, lo), jnp.where(pred, hi, mid))

    lo, _ = lax.fori_loop(0, 17, sstep,
                          (jnp.zeros((16,), jnp.int32),
                           jnp.full((16,), N, jnp.int32)))
    idx_v[pl.ds(c * 16, 16)] = lo
    return carry

  lax.fori_loop(0, _SPW // 16, qbody, 0)

  bufs = (buf0, buf1)
  gathers = [None] * _NCH
  writes = [None] * _NCH
  gathers[0] = pltpu.async_copy(x_hbm.at[idx_v.at[pl.ds(0, _CH)]], buf0, gsem)
  for g in range(_NCH):
    gathers[g].wait()
    if g + 1 < _NCH:
      if g >= 1:
        writes[g - 1].wait()
      gathers[g + 1] = pltpu.async_copy(
          x_hbm.at[idx_v.at[pl.ds((g + 1) * _CH, _CH)]],
          bufs[(g + 1) % 2], gsem)
    half = base // (N // 2)
    r0 = base - half * (N // 2) + g * _CH
    writes[g] = pltpu.async_copy(
        bufs[g % 2], out_hbm.at[pl.ds(r0, _CH), pl.ds(half * D, D)], wsem)
  writes[_NCH - 2].wait()
  writes[_NCH - 1].wait()


def _sc_sample(p_cuml, particles, kvec, plast_vec):
  mesh = plsc.VectorSubcoreMesh(core_axis_name="c", subcore_axis_name="s")
  f = pl.kernel(
      _sc_body,
      out_type=jax.ShapeDtypeStruct((N // 2, 2 * D), jnp.float32),
      mesh=mesh,
      scratch_types=[
          pltpu.VMEM((N,), jnp.float32),
          pltpu.VMEM((_SPW,), jnp.int32),
          pltpu.VMEM((32,), jnp.uint32),
          pltpu.VMEM((16,), jnp.float32),
          pltpu.VMEM((_CH, D), jnp.float32),
          pltpu.VMEM((_CH, D), jnp.float32),
          pltpu.SemaphoreType.DMA,
          pltpu.SemaphoreType.DMA,
      ],
      compiler_params=pltpu.CompilerParams(needs_layout_passes=False,
                                           use_tc_tiling_on_sc=False),
  )
  return f(p_cuml, particles, kvec, plast_vec)


_BC2 = 256


def _noise_body(k_ref, mu2_ref, lbig_ref, out_ref):
  i = pl.program_id(0)
  ks0 = k_ref[0]
  ks1 = k_ref[1]
  rows = lax.broadcasted_iota(jnp.int32, (_BC2, 2 * D), 0)
  cols = lax.broadcasted_iota(jnp.int32, (_BC2, 2 * D), 1)
  e = ((i * _BC2 + rows) * D
       + ((cols & (D - 1)) + (cols >> 6) * jnp.int32(N * D // 2)))
  bits = _random_bits_32(ks0, ks1, e.astype(jnp.uint32))
  f = _bits_to_unit_float(bits)
  u = jnp.maximum(jnp.float32(_LO), f * jnp.float32(_DELTA) + jnp.float32(_LO))
  z2 = jnp.float32(_SQRT2) * _erfinv(u)
  noise2 = lax.dot_general(z2, lbig_ref[...], (((1,), (0,)), ((), ())),
                           preferred_element_type=jnp.float32)
  out_ref[...] = jnp.float32(_OMA) * mu2_ref[...] + noise2


def _noise(kvec2, mu2, lbig):
  return pl.pallas_call(
      _noise_body,
      grid=(N // (2 * _BC2),),
      in_specs=[
          pl.BlockSpec(memory_space=pltpu.SMEM),
          pl.BlockSpec((1, 2 * D), lambda i: (0, 0)),
          pl.BlockSpec((2 * D, 2 * D), lambda i: (0, 0)),
      ],
      out_specs=pl.BlockSpec((_BC2, 2 * D), lambda i: (i, 0)),
      out_shape=jax.ShapeDtypeStruct((N // 2, 2 * D), jnp.float32),
      compiler_params=pltpu.CompilerParams(
          dimension_semantics=("arbitrary",)),
  )(kvec2, mu2, lbig)


_BF = 4096


def _final_body(chosen_ref, noise_ref, out_ref):
  h = pl.program_id(1)
  s = jnp.float32(_A) * chosen_ref[...] + noise_ref[...]
  half = jnp.where(h == 0, s[:, :D], s[:, D:])
  out_ref[...] = half.T


def _final(chosen2, noise2):
  nb = N // 2 // _BF
  return pl.pallas_call(
      _final_body,
      grid=(nb, 2),
      in_specs=[
          pl.BlockSpec((_BF, 2 * D), lambda j, h: (j, 0)),
          pl.BlockSpec((_BF, 2 * D), lambda j, h: (j, 0)),
      ],
      out_specs=pl.BlockSpec((D, _BF), lambda j, h: (0, h * nb + j)),
      out_shape=jax.ShapeDtypeStruct((D, N), jnp.float32),
      compiler_params=pltpu.CompilerParams(
          dimension_semantics=("arbitrary", "arbitrary")),
  )(chosen2, noise2)


def kernel(key, particles_locations, weights):
  key1, sub1 = jax.random.split(key)
  _, sub2 = jax.random.split(key1)
  kd1 = jax.random.key_data(sub1).astype(jnp.uint32)
  kd2 = jax.random.key_data(sub2).astype(jnp.uint32)

  p_cuml = jnp.cumsum(weights)
  plast_vec = jnp.full((16,), p_cuml[-1], jnp.float32)
  kvec = jnp.concatenate([jnp.full((16,), kd1[0], jnp.uint32),
                          jnp.full((16,), kd1[1], jnp.uint32)])

  chosen = _sc_sample(p_cuml, particles_locations, kvec, plast_vec)
  mu_col, lmat = _stats_chol(particles_locations.T, weights.reshape(1, N))
  mu = mu_col.T
  mu2 = jnp.concatenate([mu, mu], axis=1)
  lt = lmat.T
  zpad = jnp.zeros((D, D), jnp.float32)
  lbig = jnp.block([[lt, zpad], [zpad, lt]])
  noise2 = _noise(kd2, mu2, lbig)
  out_t = _final(chosen, noise2)
  out = out_t.T
  new_weights = jnp.full((N,), np.float32(1.0 / N), jnp.float32)
  return out, new_weights
